# Initial kernel scaffold; baseline (speedup 1.0000x reference)
import functools

import jax
import jax.numpy as jnp
from jax import lax
from jax.experimental import pallas as pl
from jax.experimental.pallas import tpu as pltpu
from jax.experimental.pallas import tpu_sc as plsc

N = 10000
E = 320000
D = 128

NC = 2
NS = 16
NW = NC * NS
CHUNK = 128
CPW = 79
E_PAD = NW * CPW * CHUNK
N_TAB = 10016
ZROWS = N_TAB // NS
OROWS = N // NS


def _sc_agg_builder(with_cnt):
  out_type = [jax.ShapeDtypeStruct((NC, N, D), jnp.float32)]
  if with_cnt:
    out_type.append(jax.ShapeDtypeStruct((NC, N, 16), jnp.float32))

  mesh = plsc.VectorSubcoreMesh(
      core_axis_name="c", subcore_axis_name="s", num_cores=NC, num_subcores=NS)

  scratch = [
      pltpu.VMEM((CPW, CHUNK), jnp.int32),
      pltpu.VMEM((CPW, CHUNK), jnp.int32),
      pltpu.VMEM((CHUNK, D), jnp.float32),
      pltpu.VMEM((CHUNK, 16), jnp.float32),
      pltpu.VMEM_SHARED((N_TAB, D), jnp.float32),
      pltpu.VMEM_SHARED((N_TAB, 16), jnp.float32),
      pltpu.SemaphoreType.DMA,
  ]

  def body(table_hbm, src_hbm, dst_hbm, znd_hbm, z16_hbm, ones_hbm,
           *refs):
    if with_cnt:
      p_hbm, c_hbm = refs[0], refs[1]
      rest = refs[2:]
    else:
      p_hbm = refs[0]
      rest = refs[1:]
    srcv, dstv, rows_v, ones_v, agg_s, cnt_s, sem = rest

    c = lax.axis_index("c")
    s = lax.axis_index("s")
    wid = s * NC + c

    pltpu.sync_copy(src_hbm.at[pl.ds(wid * CPW, CPW)], srcv)
    pltpu.sync_copy(dst_hbm.at[pl.ds(wid * CPW, CPW)], dstv)
    pltpu.sync_copy(znd_hbm.at[pl.ds(s * ZROWS, ZROWS)],
                    agg_s.at[pl.ds(s * ZROWS, ZROWS)])
    if with_cnt:
      pltpu.sync_copy(ones_hbm, ones_v)
      pltpu.sync_copy(z16_hbm.at[pl.ds(s * ZROWS, ZROWS)],
                      cnt_s.at[pl.ds(s * ZROWS, ZROWS)])
    plsc.subcore_barrier()

    def step(j, carry):
      pltpu.async_copy(table_hbm.at[srcv.at[j]], rows_v, sem).wait()
      pltpu.sync_copy(rows_v, agg_s.at[dstv.at[j]], add=True)
      if with_cnt:
        pltpu.sync_copy(ones_v, cnt_s.at[dstv.at[j]], add=True)
      return carry

    lax.fori_loop(0, CPW, step, 0)
    plsc.subcore_barrier()

    pltpu.sync_copy(agg_s.at[pl.ds(s * OROWS, OROWS)],
                    p_hbm.at[c, pl.ds(s * OROWS, OROWS)])
    if with_cnt:
      pltpu.sync_copy(cnt_s.at[pl.ds(s * OROWS, OROWS)],
                      c_hbm.at[c, pl.ds(s * OROWS, OROWS)])

  return pl.kernel(body, out_type=out_type, mesh=mesh, scratch_types=scratch)


_sc_agg_cnt = _sc_agg_builder(True)
_sc_agg = _sc_agg_builder(False)


def _tc_body(p_ref, c_ref, x_ref, wl_ref, b_ref, wr_ref, o_ref):
  cnt = jnp.maximum(c_ref[0, :, 0:1] + c_ref[1, :, 0:1], 1.0)
  agg = (p_ref[0] + p_ref[1]) / cnt
  o_ref[...] = jnp.maximum(
      jnp.dot(agg, wl_ref[...], preferred_element_type=jnp.float32)
      + b_ref[...]
      + jnp.dot(x_ref[...], wr_ref[...], preferred_element_type=jnp.float32),
      0.0)


def _tc_layer(p, cpart, x, wlT, b, wrT):
  R = 1000
  grid = (N // R,)
  return pl.pallas_call(
      _tc_body,
      grid=grid,
      in_specs=[
          pl.BlockSpec((NC, R, D), lambda i: (0, i, 0)),
          pl.BlockSpec((NC, R, 16), lambda i: (0, i, 0)),
          pl.BlockSpec((R, D), lambda i: (i, 0)),
          pl.BlockSpec((D, D), lambda i: (0, 0)),
          pl.BlockSpec((1, D), lambda i: (0, 0)),
          pl.BlockSpec((D, D), lambda i: (0, 0)),
      ],
      out_specs=pl.BlockSpec((R, D), lambda i: (i, 0)),
      out_shape=jax.ShapeDtypeStruct((N, D), jnp.float32),
  )(p, cpart, x, wlT, b.reshape(1, D), wrT)


def kernel(x, edge_index, W1l, b1, W1r, W2l, b2, W2r):
  src = edge_index[0].astype(jnp.int32)
  dst = edge_index[1].astype(jnp.int32)
  pad = E_PAD - E
  src2 = jnp.concatenate([src, jnp.zeros((pad,), jnp.int32)]).reshape(-1, CHUNK)
  dst2 = jnp.concatenate([dst, jnp.full((pad,), N, jnp.int32)]).reshape(-1, CHUNK)

  znd = jnp.zeros((N_TAB, D), jnp.float32)
  z16 = jnp.zeros((N_TAB, 16), jnp.float32)
  ones = jnp.ones((CHUNK, 16), jnp.float32)

  p1, cpart = _sc_agg_cnt(x, src2, dst2, znd, z16, ones)
  h = _tc_layer(p1, cpart, x, W1l.T, b1, W1r.T)
  (p2,) = _sc_agg(h, src2, dst2, znd, z16, ones)
  out = _tc_layer(p2, cpart, h, W2l.T, b2, W2r.T)
  return out


# speedup vs baseline: 3.1663x; 3.1663x over previous
import jax
import jax.numpy as jnp
from jax import lax
from jax.experimental import pallas as pl
from jax.experimental.pallas import tpu as pltpu
from jax.experimental.pallas import tpu_sc as plsc

N = 10000
E = 320000
D = 128

NC = 2
NS = 16
NW = NC * NS
CHUNK = 128
CPW = 80
E_PAD = NW * CPW * CHUNK
N_TAB = 10112
ZROWS = N_TAB // NS
GRP = 8

_MESH = plsc.VectorSubcoreMesh(
    core_axis_name="c", subcore_axis_name="s", num_cores=NC, num_subcores=NS)


def _sc_builder(kind):
  out_type = jax.ShapeDtypeStruct((NC, N_TAB, D), jnp.float32)
  scratch = [
      pltpu.VMEM((GRP, CHUNK), jnp.int32),
      pltpu.VMEM((GRP, CHUNK), jnp.int32),
      pltpu.VMEM((CHUNK, D), jnp.float32),
      pltpu.VMEM_SHARED((N_TAB, D), jnp.float32),
      pltpu.SemaphoreType.DMA,
  ]

  def body(table_hbm, src_hbm, dst_hbm, znd_hbm, p_hbm,
           srcv, dstv, rows_v, agg_s, sem):
    c = lax.axis_index("c")
    s = lax.axis_index("s")
    wid = s * NC + c

    pltpu.sync_copy(znd_hbm.at[pl.ds(s * ZROWS, ZROWS)],
                    agg_s.at[pl.ds(s * ZROWS, ZROWS)])
    if kind == "cnt":
      pltpu.sync_copy(table_hbm, rows_v)
    plsc.subcore_barrier()

    def group(g, carry):
      base = wid * CPW + g * GRP
      if kind == "agg":
        pltpu.sync_copy(src_hbm.at[pl.ds(base, GRP)], srcv)
      pltpu.sync_copy(dst_hbm.at[pl.ds(base, GRP)], dstv)

      def step(j, c2):
        if kind == "agg":
          pltpu.async_copy(table_hbm.at[srcv.at[j]], rows_v, sem).wait()
        pltpu.sync_copy(rows_v, agg_s.at[dstv.at[j]], add=True)
        return c2

      lax.fori_loop(0, GRP, step, 0)
      return carry

    lax.fori_loop(0, CPW // GRP, group, 0)
    plsc.subcore_barrier()

    pltpu.sync_copy(agg_s.at[pl.ds(s * ZROWS, ZROWS)],
                    p_hbm.at[c, pl.ds(s * ZROWS, ZROWS)])

  return pl.kernel(body, out_type=out_type, mesh=_MESH, scratch_types=scratch)


_sc_agg = _sc_builder("agg")
_sc_cnt = _sc_builder("cnt")


def _tc_body(p_ref, c_ref, x_ref, wl_ref, b_ref, wr_ref, o_ref):
  cnt = jnp.maximum(c_ref[0, :, 0:1] + c_ref[1, :, 0:1], 1.0)
  agg = (p_ref[0] + p_ref[1]) / cnt
  o_ref[...] = jnp.maximum(
      jnp.dot(agg, wl_ref[...], preferred_element_type=jnp.float32)
      + b_ref[...]
      + jnp.dot(x_ref[...], wr_ref[...], preferred_element_type=jnp.float32),
      0.0)


def _tc_layer(p, cnt128, x, wlT, b, wrT):
  R = 1000
  return pl.pallas_call(
      _tc_body,
      grid=(N // R,),
      in_specs=[
          pl.BlockSpec((NC, R, D), lambda i: (0, i, 0)),
          pl.BlockSpec((NC, R, D), lambda i: (0, i, 0)),
          pl.BlockSpec((R, D), lambda i: (i, 0)),
          pl.BlockSpec((D, D), lambda i: (0, 0)),
          pl.BlockSpec((1, D), lambda i: (0, 0)),
          pl.BlockSpec((D, D), lambda i: (0, 0)),
      ],
      out_specs=pl.BlockSpec((R, D), lambda i: (i, 0)),
      out_shape=jax.ShapeDtypeStruct((N, D), jnp.float32),
  )(p, cnt128, x, wlT, b.reshape(1, D), wrT)


def kernel(x, edge_index, W1l, b1, W1r, W2l, b2, W2r):
  src = edge_index[0].astype(jnp.int32)
  dst = edge_index[1].astype(jnp.int32)
  pad = E_PAD - E
  src2 = jnp.concatenate([src, jnp.zeros((pad,), jnp.int32)]).reshape(-1, CHUNK)
  dst2 = jnp.concatenate([dst, jnp.full((pad,), N, jnp.int32)]).reshape(-1, CHUNK)

  znd = jnp.zeros((N_TAB, D), jnp.float32)
  ones = jnp.ones((CHUNK, D), jnp.float32)

  cnt128 = _sc_cnt(ones, src2, dst2, znd)
  p1 = _sc_agg(x, src2, dst2, znd)
  h = _tc_layer(p1, cnt128, x, W1l.T, b1, W1r.T)
  p2 = _sc_agg(h, src2, dst2, znd)
  out = _tc_layer(p2, cnt128, h, W2l.T, b2, W2r.T)
  return out


# speedup vs baseline: 3.4195x; 1.0800x over previous
import jax
import jax.numpy as jnp
from jax import lax
from jax.experimental import pallas as pl
from jax.experimental.pallas import tpu as pltpu
from jax.experimental.pallas import tpu_sc as plsc

N = 10000
E = 320000
D = 128

NC = 2
NS = 16
NW = NC * NS
CHUNK = 128
CPW = 80
E_PAD = NW * CPW * CHUNK
N_TAB = 10112
ZROWS = N_TAB // NS
GRP = 8

_MESH = plsc.VectorSubcoreMesh(
    core_axis_name="c", subcore_axis_name="s", num_cores=NC, num_subcores=NS)


def _sc_builder(kind):
  out_type = jax.ShapeDtypeStruct((NC, N_TAB, D), jnp.float32)
  scratch = [
      pltpu.VMEM((GRP, CHUNK), jnp.int32),
      pltpu.VMEM((GRP, CHUNK), jnp.int32),
      pltpu.VMEM((CHUNK, D), jnp.float32),
      pltpu.VMEM((CHUNK, D), jnp.float32),
      pltpu.VMEM_SHARED((N_TAB, D), jnp.float32),
      pltpu.SemaphoreType.DMA,
      pltpu.SemaphoreType.DMA,
  ]

  def body(table_hbm, src_hbm, dst_hbm, znd_hbm, p_hbm,
           srcv, dstv, rows_a, rows_b, agg_s, sem_a, sem_b):
    c = lax.axis_index("c")
    s = lax.axis_index("s")
    wid = s * NC + c
    bufs = [rows_a, rows_b]
    sems = [sem_a, sem_b]

    pltpu.sync_copy(znd_hbm.at[pl.ds(s * ZROWS, ZROWS)],
                    agg_s.at[pl.ds(s * ZROWS, ZROWS)])
    if kind == "cnt":
      pltpu.sync_copy(table_hbm, rows_a)
    plsc.subcore_barrier()

    def group(g, carry):
      base = wid * CPW + g * GRP
      if kind == "agg":
        pltpu.sync_copy(src_hbm.at[pl.ds(base, GRP)], srcv)
      pltpu.sync_copy(dst_hbm.at[pl.ds(base, GRP)], dstv)

      if kind == "agg":
        descs = [None, None]
        descs[0] = pltpu.async_copy(table_hbm.at[srcv.at[0]], bufs[0], sems[0])
        for j in range(GRP):
          if j + 1 < GRP:
            descs[(j + 1) % 2] = pltpu.async_copy(
                table_hbm.at[srcv.at[j + 1]], bufs[(j + 1) % 2],
                sems[(j + 1) % 2])
          descs[j % 2].wait()
          pltpu.sync_copy(bufs[j % 2], agg_s.at[dstv.at[j]], add=True)
      else:
        for j in range(GRP):
          pltpu.sync_copy(rows_a, agg_s.at[dstv.at[j]], add=True)
      return carry

    lax.fori_loop(0, CPW // GRP, group, 0)
    plsc.subcore_barrier()

    pltpu.sync_copy(agg_s.at[pl.ds(s * ZROWS, ZROWS)],
                    p_hbm.at[c, pl.ds(s * ZROWS, ZROWS)])

  return pl.kernel(body, out_type=out_type, mesh=_MESH, scratch_types=scratch)


_sc_agg = _sc_builder("agg")
_sc_cnt = _sc_builder("cnt")


def _tc_body(p_ref, c_ref, x_ref, wl_ref, b_ref, wr_ref, o_ref):
  cnt = jnp.maximum(c_ref[0, :, 0:1] + c_ref[1, :, 0:1], 1.0)
  agg = (p_ref[0] + p_ref[1]) / cnt
  o_ref[...] = jnp.maximum(
      jnp.dot(agg, wl_ref[...], preferred_element_type=jnp.float32)
      + b_ref[...]
      + jnp.dot(x_ref[...], wr_ref[...], preferred_element_type=jnp.float32),
      0.0)


def _tc_layer(p, cnt128, x, wlT, b, wrT):
  R = 1000
  return pl.pallas_call(
      _tc_body,
      grid=(N // R,),
      in_specs=[
          pl.BlockSpec((NC, R, D), lambda i: (0, i, 0)),
          pl.BlockSpec((NC, R, D), lambda i: (0, i, 0)),
          pl.BlockSpec((R, D), lambda i: (i, 0)),
          pl.BlockSpec((D, D), lambda i: (0, 0)),
          pl.BlockSpec((1, D), lambda i: (0, 0)),
          pl.BlockSpec((D, D), lambda i: (0, 0)),
      ],
      out_specs=pl.BlockSpec((R, D), lambda i: (i, 0)),
      out_shape=jax.ShapeDtypeStruct((N, D), jnp.float32),
  )(p, cnt128, x, wlT, b.reshape(1, D), wrT)


def kernel(x, edge_index, W1l, b1, W1r, W2l, b2, W2r):
  src = edge_index[0].astype(jnp.int32)
  dst = edge_index[1].astype(jnp.int32)
  pad = E_PAD - E
  src2 = jnp.concatenate([src, jnp.zeros((pad,), jnp.int32)]).reshape(-1, CHUNK)
  dst2 = jnp.concatenate([dst, jnp.full((pad,), N, jnp.int32)]).reshape(-1, CHUNK)

  znd = jnp.zeros((N_TAB, D), jnp.float32)
  ones = jnp.ones((CHUNK, D), jnp.float32)

  cnt128 = _sc_cnt(ones, src2, dst2, znd)
  p1 = _sc_agg(x, src2, dst2, znd)
  h = _tc_layer(p1, cnt128, x, W1l.T, b1, W1r.T)
  p2 = _sc_agg(h, src2, dst2, znd)
  out = _tc_layer(p2, cnt128, h, W2l.T, b2, W2r.T)
  return out


# speedup vs baseline: 3.8349x; 1.1215x over previous
import jax
import jax.numpy as jnp
from jax import lax
from jax.experimental import pallas as pl
from jax.experimental.pallas import tpu as pltpu
from jax.experimental.pallas import tpu_sc as plsc

N = 10000
E = 320000
D = 128

NC = 2
NS = 16
NW = NC * NS
CHUNK = 128
CPW0 = 128
CPW1 = 32
E_PAD = NS * (CPW0 + CPW1) * CHUNK
N_TAB = 10112
ZROWS = N_TAB // NS
GRP = 8

_MESH = plsc.VectorSubcoreMesh(
    core_axis_name="c", subcore_axis_name="s", num_cores=NC, num_subcores=NS)


def _sc_builder(kind, cpw0, cpw1):
  out_type = jax.ShapeDtypeStruct((NC, N_TAB, D), jnp.float32)
  scratch = [
      pltpu.VMEM((GRP, CHUNK), jnp.int32),
      pltpu.VMEM((GRP, CHUNK), jnp.int32),
      pltpu.VMEM((CHUNK, D), jnp.float32),
      pltpu.VMEM((CHUNK, D), jnp.float32),
      pltpu.VMEM_SHARED((N_TAB, D), jnp.float32),
      pltpu.SemaphoreType.DMA,
      pltpu.SemaphoreType.DMA,
  ]

  def body(table_hbm, src_hbm, dst_hbm, znd_hbm, p_hbm,
           srcv, dstv, rows_a, rows_b, agg_s, sem_a, sem_b):
    c = lax.axis_index("c")
    s = lax.axis_index("s")
    wid = s * NC + c
    bufs = [rows_a, rows_b]
    sems = [sem_a, sem_b]

    pltpu.sync_copy(znd_hbm.at[pl.ds(s * ZROWS, ZROWS)],
                    agg_s.at[pl.ds(s * ZROWS, ZROWS)])
    if kind == "cnt":
      pltpu.sync_copy(table_hbm, rows_a)
    plsc.subcore_barrier()

    def group_at(start):
      def group(g, carry):
        base = start + g * GRP
        if kind == "agg":
          pltpu.sync_copy(src_hbm.at[pl.ds(base, GRP)], srcv)
        pltpu.sync_copy(dst_hbm.at[pl.ds(base, GRP)], dstv)

        if kind == "agg":
          descs = [None, None]
          descs[0] = pltpu.async_copy(table_hbm.at[srcv.at[0]], bufs[0],
                                      sems[0])
          for j in range(GRP):
            if j + 1 < GRP:
              descs[(j + 1) % 2] = pltpu.async_copy(
                  table_hbm.at[srcv.at[j + 1]], bufs[(j + 1) % 2],
                  sems[(j + 1) % 2])
            descs[j % 2].wait()
            pltpu.sync_copy(bufs[j % 2], agg_s.at[dstv.at[j]], add=True)
        else:
          for j in range(GRP):
            pltpu.sync_copy(rows_a, agg_s.at[dstv.at[j]], add=True)
        return carry
      return group

    if cpw0 == cpw1:
      lax.fori_loop(0, cpw0 // GRP, group_at(wid * cpw0), 0)
    else:
      @pl.when(c == 0)
      def _():
        lax.fori_loop(0, cpw0 // GRP, group_at(s * cpw0), 0)

      @pl.when(c == 1)
      def _():
        lax.fori_loop(0, cpw1 // GRP, group_at(NS * cpw0 + s * cpw1), 0)
    plsc.subcore_barrier()

    pltpu.sync_copy(agg_s.at[pl.ds(s * ZROWS, ZROWS)],
                    p_hbm.at[c, pl.ds(s * ZROWS, ZROWS)])

  return pl.kernel(body, out_type=out_type, mesh=_MESH, scratch_types=scratch)


_sc_agg = _sc_builder("agg", CPW0, CPW1)
_sc_cnt = _sc_builder("cnt", (CPW0 + CPW1) // 2, (CPW0 + CPW1) // 2)


def _tc_body(p_ref, c_ref, x_ref, wl_ref, b_ref, wr_ref, o_ref):
  cnt = jnp.maximum(c_ref[0, :, 0:1] + c_ref[1, :, 0:1], 1.0)
  agg = (p_ref[0] + p_ref[1]) / cnt
  o_ref[...] = jnp.maximum(
      jnp.dot(agg, wl_ref[...], preferred_element_type=jnp.float32)
      + b_ref[...]
      + jnp.dot(x_ref[...], wr_ref[...], preferred_element_type=jnp.float32),
      0.0)


def _tc_layer(p, cnt128, x, wlT, b, wrT):
  R = 1000
  return pl.pallas_call(
      _tc_body,
      grid=(N // R,),
      in_specs=[
          pl.BlockSpec((NC, R, D), lambda i: (0, i, 0)),
          pl.BlockSpec((NC, R, D), lambda i: (0, i, 0)),
          pl.BlockSpec((R, D), lambda i: (i, 0)),
          pl.BlockSpec((D, D), lambda i: (0, 0)),
          pl.BlockSpec((1, D), lambda i: (0, 0)),
          pl.BlockSpec((D, D), lambda i: (0, 0)),
      ],
      out_specs=pl.BlockSpec((R, D), lambda i: (i, 0)),
      out_shape=jax.ShapeDtypeStruct((N, D), jnp.float32),
  )(p, cnt128, x, wlT, b.reshape(1, D), wrT)


def kernel(x, edge_index, W1l, b1, W1r, W2l, b2, W2r):
  src = edge_index[0].astype(jnp.int32)
  dst = edge_index[1].astype(jnp.int32)
  pad = E_PAD - E
  src2 = jnp.concatenate([src, jnp.zeros((pad,), jnp.int32)]).reshape(-1, CHUNK)
  dst2 = jnp.concatenate([dst, jnp.full((pad,), N, jnp.int32)]).reshape(-1, CHUNK)

  znd = jnp.zeros((N_TAB, D), jnp.float32)
  ones = jnp.ones((CHUNK, D), jnp.float32)

  cnt128 = _sc_cnt(ones, src2, dst2, znd)
  p1 = _sc_agg(x, src2, dst2, znd)
  h = _tc_layer(p1, cnt128, x, W1l.T, b1, W1r.T)
  p2 = _sc_agg(h, src2, dst2, znd)
  out = _tc_layer(p2, cnt128, h, W2l.T, b2, W2r.T)
  return out


# speedup vs baseline: 3.8583x; 1.0061x over previous
import jax
import jax.numpy as jnp
from jax import lax
from jax.experimental import pallas as pl
from jax.experimental.pallas import tpu as pltpu
from jax.experimental.pallas import tpu_sc as plsc

N = 10000
E = 320000
D = 128

NC = 2
NS = 16
CHUNK = 128
NCHUNK = 2560
E_PAD = NCHUNK * CHUNK
CPW_ALL = NCHUNK // NS
CPW0 = 144
CPW1 = 16
N_TAB = 10112
ZROWS = N_TAB // NS
GRP = 8

_MESH = plsc.VectorSubcoreMesh(
    core_axis_name="c", subcore_axis_name="s", num_cores=NC, num_subcores=NS)


def _sc_builder(mode):
  out_type = jax.ShapeDtypeStruct((NC, N_TAB, D), jnp.float32)
  scratch = [
      pltpu.VMEM((GRP, CHUNK), jnp.int32),
      pltpu.VMEM((GRP, CHUNK), jnp.int32),
      pltpu.VMEM((CHUNK, D), jnp.float32),
      pltpu.VMEM((CHUNK, D), jnp.float32),
      pltpu.VMEM_SHARED((N_TAB, D), jnp.float32),
      pltpu.SemaphoreType.DMA,
      pltpu.SemaphoreType.DMA,
  ]

  def body(table_hbm, ones_hbm, src_hbm, dst_hbm, znd_hbm, p_hbm,
           srcv, dstv, rows_a, rows_b, agg_s, sem_a, sem_b):
    c = lax.axis_index("c")
    s = lax.axis_index("s")
    bufs = [rows_a, rows_b]
    sems = [sem_a, sem_b]

    pltpu.sync_copy(znd_hbm.at[pl.ds(s * ZROWS, ZROWS)],
                    agg_s.at[pl.ds(s * ZROWS, ZROWS)])
    if mode == "l1":
      @pl.when(c == 1)
      def _():
        pltpu.sync_copy(ones_hbm, rows_a)
    plsc.subcore_barrier()

    def agg_group(start):
      def group(g, carry):
        base = start + g * GRP
        pltpu.sync_copy(src_hbm.at[pl.ds(base, GRP)], srcv)
        pltpu.sync_copy(dst_hbm.at[pl.ds(base, GRP)], dstv)
        descs = [None, None]
        descs[0] = pltpu.async_copy(table_hbm.at[srcv.at[0]], bufs[0], sems[0])
        for j in range(GRP):
          if j + 1 < GRP:
            descs[(j + 1) % 2] = pltpu.async_copy(
                table_hbm.at[srcv.at[j + 1]], bufs[(j + 1) % 2],
                sems[(j + 1) % 2])
          descs[j % 2].wait()
          pltpu.sync_copy(bufs[j % 2], agg_s.at[dstv.at[j]], add=True)
        return carry
      return group

    def cnt_group(start):
      def group(g, carry):
        base = start + g * GRP
        pltpu.sync_copy(dst_hbm.at[pl.ds(base, GRP)], dstv)
        for j in range(GRP):
          pltpu.sync_copy(rows_a, agg_s.at[dstv.at[j]], add=True)
        return carry
      return group

    if mode == "l1":
      @pl.when(c == 0)
      def _():
        lax.fori_loop(0, CPW_ALL // GRP, agg_group(s * CPW_ALL), 0)

      @pl.when(c == 1)
      def _():
        lax.fori_loop(0, CPW_ALL // GRP, cnt_group(s * CPW_ALL), 0)
    else:
      @pl.when(c == 0)
      def _():
        lax.fori_loop(0, CPW0 // GRP, agg_group(s * CPW0), 0)

      @pl.when(c == 1)
      def _():
        lax.fori_loop(0, CPW1 // GRP, agg_group(NS * CPW0 + s * CPW1), 0)

    plsc.subcore_barrier()
    pltpu.sync_copy(agg_s.at[pl.ds(s * ZROWS, ZROWS)],
                    p_hbm.at[c, pl.ds(s * ZROWS, ZROWS)])

  return pl.kernel(body, out_type=out_type, mesh=_MESH, scratch_types=scratch)


_sc_l1 = _sc_builder("l1")
_sc_l2 = _sc_builder("l2")


def _tc1_body(pc_ref, x_ref, wl_ref, b_ref, wr_ref, o_ref):
  cnt = jnp.maximum(pc_ref[1, :, 0:1], 1.0)
  agg = pc_ref[0] / cnt
  o_ref[...] = jnp.maximum(
      jnp.dot(agg, wl_ref[...], preferred_element_type=jnp.float32)
      + b_ref[...]
      + jnp.dot(x_ref[...], wr_ref[...], preferred_element_type=jnp.float32),
      0.0)


def _tc2_body(p2_ref, cnt_ref, x_ref, wl_ref, b_ref, wr_ref, o_ref):
  cnt = jnp.maximum(cnt_ref[0, :, 0:1], 1.0)
  agg = (p2_ref[0] + p2_ref[1]) / cnt
  o_ref[...] = jnp.maximum(
      jnp.dot(agg, wl_ref[...], preferred_element_type=jnp.float32)
      + b_ref[...]
      + jnp.dot(x_ref[...], wr_ref[...], preferred_element_type=jnp.float32),
      0.0)


_R = 1000
_COMMON_SPECS = [
    pl.BlockSpec((_R, D), lambda i: (i, 0)),
    pl.BlockSpec((D, D), lambda i: (0, 0)),
    pl.BlockSpec((1, D), lambda i: (0, 0)),
    pl.BlockSpec((D, D), lambda i: (0, 0)),
]
_OUT_SPEC = pl.BlockSpec((_R, D), lambda i: (i, 0))


def _tc_layer1(pc, x, wlT, b, wrT):
  return pl.pallas_call(
      _tc1_body,
      grid=(N // _R,),
      in_specs=[pl.BlockSpec((NC, _R, D), lambda i: (0, i, 0))] + _COMMON_SPECS,
      out_specs=_OUT_SPEC,
      out_shape=jax.ShapeDtypeStruct((N, D), jnp.float32),
  )(pc, x, wlT, b.reshape(1, D), wrT)


def _tc_layer2(p2, pc, x, wlT, b, wrT):
  return pl.pallas_call(
      _tc2_body,
      grid=(N // _R,),
      in_specs=[pl.BlockSpec((NC, _R, D), lambda i: (0, i, 0)),
                pl.BlockSpec((1, _R, D), lambda i: (1, i, 0))] + _COMMON_SPECS,
      out_specs=_OUT_SPEC,
      out_shape=jax.ShapeDtypeStruct((N, D), jnp.float32),
  )(p2, pc, x, wlT, b.reshape(1, D), wrT)


def kernel(x, edge_index, W1l, b1, W1r, W2l, b2, W2r):
  src = edge_index[0].astype(jnp.int32)
  dst = edge_index[1].astype(jnp.int32)
  pad = E_PAD - E
  src2 = jnp.concatenate([src, jnp.zeros((pad,), jnp.int32)]).reshape(-1, CHUNK)
  dst2 = jnp.concatenate([dst, jnp.full((pad,), N, jnp.int32)]).reshape(-1, CHUNK)

  znd = jnp.zeros((N_TAB, D), jnp.float32)
  ones = jnp.ones((CHUNK, D), jnp.float32)

  pc = _sc_l1(x, ones, src2, dst2, znd)
  h = _tc_layer1(pc, x, W1l.T, b1, W1r.T)
  p2 = _sc_l2(h, ones, src2, dst2, znd)
  out = _tc_layer2(p2, pc, h, W2l.T, b2, W2r.T)
  return out
